# baseline (speedup 1.0000x reference)
import functools

import jax
import jax.numpy as jnp
from jax import lax
from jax.experimental import pallas as pl
from jax.experimental.pallas import tpu as pltpu
from jax.experimental.pallas import tpu_sc as plsc

B = 512
NPG = 68
N = B * NPG
E = N * 16
FH = 128
C = 16
EPS = 1e-15

NC, NS, LANES = 2, 16, 16
NW = NC * NS
NB = 64
BR = N // NB
CAP = 544
EPT = E // NW
ECH = 4352
GB = 64
MULT, MSH = 61681, 25
DUMMY = BR << 16


def _c(v):
    return jnp.int32(v)


def _fori(n, body, unroll=None):
    if isinstance(n, int):
        return lax.fori_loop(0, n, body, 0, unroll=unroll)
    return lax.fori_loop(_c(0), n.astype(jnp.int32), body, 0)


def _mo(v, m):
    return pl.multiple_of(v, m)


def _wid():
    return (lax.axis_index("s").astype(jnp.int32) * _c(NC)
            + lax.axis_index("c").astype(jnp.int32))


def _vperm(v, idx):
    dnums = lax.GatherDimensionNumbers(
        offset_dims=(), collapsed_slice_dims=(0,), start_index_map=(0,))
    return lax.gather(v, idx[:, None], dnums, slice_sizes=(1,),
                      mode=lax.GatherScatterMode.PROMISE_IN_BOUNDS)


def _i32(v):
    return jnp.full((LANES,), v, jnp.int32)


def _bin_body(src_hbm, dst_hbm, bins_hbm, cnts_hbm, sbuf, dbuf, binsv, cntv):
    w = _wid()
    iota = lax.iota(jnp.int32, LANES)
    ones = _i32(1)
    dummy16 = _i32(DUMMY)
    zero16 = _i32(0)

    def initb(i, _):
        binsv[pl.ds(_mo(i * _c(16), 16), 16)] = dummy16
        return 0

    _fori(NB * CAP // 16, initb, unroll=8)
    for j in range(NB // 16):
        cntv[pl.ds(j * 16, 16)] = zero16

    shifts = [(jnp.maximum(iota - k, 0), iota >= k) for k in range(1, 16)]

    def edge_vreg(i, _):
        s16 = sbuf[pl.ds(_mo(i * _c(16), 16), 16)]
        d16 = dbuf[pl.ds(_mo(i * _c(16), 16), 16)]
        bkt = (d16 * MULT) >> MSH
        ldst = d16 - bkt * BR
        packed = s16 | (ldst << 16)
        base = plsc.load_gather(cntv, [bkt])
        rank = zero16
        for idxk, mk in shifts:
            sh = _vperm(bkt, idxk)
            rank = rank + jnp.where(mk & (sh == bkt), 1, 0).astype(jnp.int32)
        pos = jnp.minimum(base + rank, CAP - 1)
        plsc.store_scatter(binsv, [bkt * CAP + pos], packed)
        plsc.addupdate_scatter(cntv, [bkt], ones)
        return 0

    for ch in range(EPT // ECH):
        base_off = _mo(w * _c(EPT) + _c(ch * ECH), 64)
        pltpu.sync_copy(src_hbm.at[pl.ds(base_off, ECH)], sbuf)
        pltpu.sync_copy(dst_hbm.at[pl.ds(base_off, ECH)], dbuf)
        _fori(ECH // 16, edge_vreg)

    pltpu.sync_copy(binsv, bins_hbm.at[pl.ds(_mo(w * _c(NB * CAP), 16), NB * CAP)])
    pltpu.sync_copy(cntv, cnts_hbm.at[pl.ds(_mo(w * _c(NB), 16), NB)])


def _count_at(cbuf, iota, s, b):
    j = s * _c(NB) + b
    vec = cbuf[pl.ds(_mo((j >> _c(4)) << _c(4), 16), 16)]
    return jnp.sum(jnp.where(iota == (j & _c(15)), vec, 0).astype(jnp.int32),
                   dtype=jnp.int32)


def _agg_body(htab, bins_hbm, cnts_hbm, agg_hbm,
              acc, merged, idx0, idx1, rows0, rows1, cbuf,
              sem0, sem1, sems):
    w = _wid()
    iota = lax.iota(jnp.int32, LANES)
    pltpu.sync_copy(cnts_hbm, cbuf)
    zf = jnp.zeros((LANES,), jnp.float32)
    dummy16 = _i32(DUMMY)

    def zrow(i, _):
        acc[pl.ds(_mo(i * _c(16), 16), 16)] = zf
        return 0

    def prep(k, idxr):
        for g in range(GB // 16):
            pk = merged[pl.ds(_mo(k * _c(GB) + _c(g * 16), 16), 16)]
            idxr[pl.ds(g * 16, 16)] = pk & 0xFFFF

    rbase = [iota + _c(g * 16) for g in range(GB // 16)]

    def accum(k, rowsr):
        for g in range(GB // 16):
            pk = merged[pl.ds(_mo(k * _c(GB) + _c(g * 16), 16), 16)]
            aidx0 = (pk >> 16) << 7
            rowv = rbase[g]

            def colb(c, carry):
                colv, aidx = carry
                v = plsc.load_gather(rowsr, [rowv, colv])
                plsc.addupdate_scatter(acc, [aidx], v)
                return (colv + 1, aidx + 1)

            lax.fori_loop(0, FH, colb, (jnp.zeros((LANES,), jnp.int32), aidx0),
                          unroll=16)

    for r in range(NB // NW):
        b = _c(r * NW) + w
        _fori((BR + 1) * FH // 16, zrow, unroll=8)

        descs = []
        for s in range(NW):
            d = pltpu.async_copy(
                bins_hbm.at[pl.ds(_mo(_c(s * NB * CAP) + b * _c(CAP), 16),
                                  CAP)],
                merged.at[pl.ds(s * CAP, CAP)], sems)
            descs.append(d)
        for d in descs:
            d.wait()

        cursor = _c(0)
        for s in range(NW):
            cnt = _count_at(cbuf, iota, _c(s), b)
            nv = (cnt + _c(15)) >> _c(4)
            cur_s = cursor

            def cp(k2, _, cur_s=cur_s, s=s):
                v = merged[pl.ds(_mo(_c(s * CAP) + k2 * _c(16), 16), 16)]
                merged[pl.ds(_mo(cur_s + k2 * _c(16), 16), 16)] = v
                return 0

            if s > 0:
                _fori(nv, cp)
            cursor = cursor + ((cnt + _c(15)) & _c(-16))

        for t in range(GB // 16):
            merged[pl.ds(_mo(cursor + _c(t * 16), 16), 16)] = dummy16

        nbm = (cursor + _c(GB - 1)) >> _c(6)
        npair = (nbm + _c(1)) >> _c(1)

        @pl.when(nbm > 0)
        def _():
            prep(_c(0), idx0)
            pltpu.async_copy(htab.at[idx0], rows0, sem0)

        def pair(pp, _):
            k0 = pp * _c(2)
            k1 = k0 + _c(1)

            @pl.when(k1 < nbm)
            def _():
                prep(k1, idx1)
                pltpu.async_copy(htab.at[idx1], rows1, sem1)

            pltpu.make_async_copy(
                htab.at[idx0], rows0, sem0).wait()
            accum(k0, rows0)

            @pl.when(k1 + _c(1) < nbm)
            def _():
                prep(k1 + _c(1), idx0)
                pltpu.async_copy(htab.at[idx0], rows0, sem0)

            @pl.when(k1 < nbm)
            def _():
                pltpu.make_async_copy(
                    htab.at[idx1], rows1, sem1).wait()
                accum(k1, rows1)

            return 0

        _fori(npair, pair)
        pltpu.sync_copy(acc.at[pl.ds(0, BR * FH)],
                        agg_hbm.at[pl.ds(_mo(b * _c(BR * FH), 16), BR * FH)])


def _agg1_body(xflat_hbm, bins_hbm, cnts_hbm, agg_hbm,
               xbuf, acc1, pbuf, cbuf):
    w = _wid()
    iota = lax.iota(jnp.int32, LANES)
    pltpu.sync_copy(cnts_hbm, cbuf)
    pltpu.sync_copy(xflat_hbm, xbuf)
    zf = jnp.zeros((LANES,), jnp.float32)

    def zv(i, _):
        acc1[pl.ds(_mo(i * _c(16), 16), 16)] = zf
        return 0

    for r in range(NB // NW):
        b = _c(r * NW) + w
        _fori(1104 // 16, zv)

        def src_list(s, _):
            cnt = _count_at(cbuf, iota, s, b)
            nv = (cnt + _c(15)) >> _c(4)
            pltpu.sync_copy(bins_hbm.at[pl.ds(_mo(s * _c(NB * CAP) + b * _c(CAP), 16), CAP)], pbuf)

            def vreg(k, _):
                pk = pbuf[pl.ds(_mo(k * _c(16), 16), 16)]
                s2 = (pk & 0xFFFF) * 2
                d2 = (pk >> 16) * 2
                v0 = plsc.load_gather(xbuf, [s2])
                v1 = plsc.load_gather(xbuf, [s2 + 1])
                plsc.addupdate_scatter(acc1, [d2], v0)
                plsc.addupdate_scatter(acc1, [d2 + 1], v1)
                return 0

            _fori(nv, vreg)
            return 0

        _fori(NW, src_list)
        pltpu.sync_copy(acc1.at[pl.ds(0, BR * 2)],
                        agg_hbm.at[pl.ds(_mo(b * _c(BR * 2), 16), BR * 2)])


def _sc_binning(src, dst):
    mesh = plsc.VectorSubcoreMesh(core_axis_name="c", subcore_axis_name="s")
    f = pl.kernel(
        _bin_body,
        compiler_params=pltpu.CompilerParams(needs_layout_passes=False),
        out_type=(
            jax.ShapeDtypeStruct((NW * NB * CAP,), jnp.int32),
            jax.ShapeDtypeStruct((NW * NB,), jnp.int32),
        ),
        mesh=mesh,
        scratch_types=[
            pltpu.VMEM((ECH,), jnp.int32),
            pltpu.VMEM((ECH,), jnp.int32),
            pltpu.VMEM((NB * CAP,), jnp.int32),
            pltpu.VMEM((NB,), jnp.int32),
        ],
    )
    return f(src, dst)


def _sc_agg128(htab, bins, cnts):
    mesh = plsc.VectorSubcoreMesh(core_axis_name="c", subcore_axis_name="s")
    f = pl.kernel(
        _agg_body,
        compiler_params=pltpu.CompilerParams(needs_layout_passes=False),
        out_type=jax.ShapeDtypeStruct((N * FH,), jnp.float32),
        mesh=mesh,
        scratch_types=[
            pltpu.VMEM(((BR + 1) * FH,), jnp.float32),
            pltpu.VMEM((NW * CAP + GB,), jnp.int32),
            pltpu.VMEM((GB,), jnp.int32),
            pltpu.VMEM((GB,), jnp.int32),
            pltpu.VMEM((GB, FH), jnp.float32),
            pltpu.VMEM((GB, FH), jnp.float32),
            pltpu.VMEM((NW * NB,), jnp.int32),
            pltpu.SemaphoreType.DMA,
            pltpu.SemaphoreType.DMA,
            pltpu.SemaphoreType.DMA,
        ],
    )
    return f(htab, bins, cnts).reshape(N, FH)


def _sc_agg1(xflat, bins, cnts):
    mesh = plsc.VectorSubcoreMesh(core_axis_name="c", subcore_axis_name="s")
    f = pl.kernel(
        _agg1_body,
        compiler_params=pltpu.CompilerParams(needs_layout_passes=False),
        out_type=jax.ShapeDtypeStruct((N * 2,), jnp.float32),
        mesh=mesh,
        scratch_types=[
            pltpu.VMEM((N * 2,), jnp.float32),
            pltpu.VMEM((1104,), jnp.float32),
            pltpu.VMEM((CAP,), jnp.int32),
            pltpu.VMEM((NW * NB,), jnp.int32),
        ],
    )
    return f(xflat, bins, cnts)


_RB = 512


def _l1_body(xc_ref, w_ref, b_ref, out_ref):
    z = jnp.dot(xc_ref[...], w_ref[...], preferred_element_type=jnp.float32)
    out_ref[...] = jnp.maximum(z + b_ref[0:1, :], 0.0)


def _mm_body(agg_ref, h_ref, wr_ref, wro_ref, b_ref, out_ref):
    z = (jnp.dot(agg_ref[...], wr_ref[...], preferred_element_type=jnp.float32)
         + jnp.dot(h_ref[...], wro_ref[...], preferred_element_type=jnp.float32))
    out_ref[...] = jnp.maximum(z + b_ref[0:1, :], 0.0)


def _tc_layer1(xcat8, w8, bias8):
    grid = (N // _RB,)
    return pl.pallas_call(
        _l1_body,
        grid=grid,
        in_specs=[
            pl.BlockSpec((_RB, 8), lambda i: (i, 0)),
            pl.BlockSpec((8, FH), lambda i: (0, 0)),
            pl.BlockSpec((8, FH), lambda i: (0, 0)),
        ],
        out_specs=pl.BlockSpec((_RB, FH), lambda i: (i, 0)),
        out_shape=jax.ShapeDtypeStruct((N, FH), jnp.float32),
    )(xcat8, w8, bias8)


def _tc_layer(agg, h, wrT, wroT, bias8):
    grid = (N // _RB,)
    return pl.pallas_call(
        _mm_body,
        grid=grid,
        in_specs=[
            pl.BlockSpec((_RB, FH), lambda i: (i, 0)),
            pl.BlockSpec((_RB, FH), lambda i: (i, 0)),
            pl.BlockSpec((FH, FH), lambda i: (0, 0)),
            pl.BlockSpec((FH, FH), lambda i: (0, 0)),
            pl.BlockSpec((8, FH), lambda i: (0, 0)),
        ],
        out_specs=pl.BlockSpec((_RB, FH), lambda i: (i, 0)),
        out_shape=jax.ShapeDtypeStruct((N, FH), jnp.float32),
    )(agg, h, wrT, wroT, bias8)


_BB = 64


def _pool_body(s_ref, adj_ref, xr_ref, out_ref, oadj_ref, link_ref, ent_ref):
    i = pl.program_id(0)
    sb = s_ref[...]
    m = jnp.max(sb, axis=-1, keepdims=True)
    e = jnp.exp(sb - m)
    ss = e / jnp.sum(e, axis=-1, keepdims=True)
    xr = xr_ref[...]
    out_ref[...] = lax.dot_general(
        ss, xr, (((1,), (1,)), ((0,), (0,))), preferred_element_type=jnp.float32)
    adjb = adj_ref[...]
    asx = lax.dot_general(
        adjb, ss, (((2,), (1,)), ((0,), (0,))), preferred_element_type=jnp.float32)
    oadj_ref[...] = lax.dot_general(
        ss, asx, (((1,), (1,)), ((0,), (0,))), preferred_element_type=jnp.float32)
    sst = lax.dot_general(
        ss, ss, (((2,), (2,)), ((0,), (0,))), preferred_element_type=jnp.float32)
    link = adjb - sst
    lpart = jnp.sum(link * link)
    epart = jnp.sum(-ss * jnp.log(ss + EPS))
    r0 = lax.broadcasted_iota(jnp.int32, (8, 128), 0)
    c0 = lax.broadcasted_iota(jnp.int32, (8, 128), 1)
    mask00 = (r0 == 0) & (c0 == 0)

    @pl.when(i == 0)
    def _():
        link_ref[...] = jnp.zeros((8, 128), jnp.float32)
        ent_ref[...] = jnp.zeros((8, 128), jnp.float32)

    link_ref[...] = link_ref[...] + jnp.where(mask00, lpart, 0.0)
    ent_ref[...] = ent_ref[...] + jnp.where(mask00, epart, 0.0)

    @pl.when(i == pl.num_programs(0) - 1)
    def _():
        lv = link_ref[...]
        link_ref[...] = jnp.where(
            mask00, jnp.sqrt(lv) / float(B * NPG * NPG), lv)
        ev = ent_ref[...]
        ent_ref[...] = jnp.where(mask00, ev / float(B * NPG), ev)


def _tc_pool(s, adj, xr):
    grid = (B // _BB,)
    return pl.pallas_call(
        _pool_body,
        grid=grid,
        in_specs=[
            pl.BlockSpec((_BB, NPG, C), lambda i: (i, 0, 0)),
            pl.BlockSpec((_BB, NPG, NPG), lambda i: (i, 0, 0)),
            pl.BlockSpec((_BB, NPG, FH), lambda i: (i, 0, 0)),
        ],
        out_specs=[
            pl.BlockSpec((_BB, C, FH), lambda i: (i, 0, 0)),
            pl.BlockSpec((_BB, C, C), lambda i: (i, 0, 0)),
            pl.BlockSpec((8, 128), lambda i: (0, 0)),
            pl.BlockSpec((8, 128), lambda i: (0, 0)),
        ],
        out_shape=[
            jax.ShapeDtypeStruct((B, C, FH), jnp.float32),
            jax.ShapeDtypeStruct((B, C, C), jnp.float32),
            jax.ShapeDtypeStruct((8, 128), jnp.float32),
            jax.ShapeDtypeStruct((8, 128), jnp.float32),
        ],
    )(s, adj, xr)


def kernel(x, edge_index, adj, s, pos,
           W1_rel, b1, W1_root, W2_rel, b2, W2_root,
           W3_rel, b3, W3_root):
    with jax.enable_x64(False):
        return _kernel32(x, edge_index, adj, s, pos,
                         W1_rel, b1, W1_root, W2_rel, b2, W2_root,
                         W3_rel, b3, W3_root)


def _kernel32(x, edge_index, adj, s, pos,
              W1_rel, b1, W1_root, W2_rel, b2, W2_root,
              W3_rel, b3, W3_root):
    f32 = jnp.float32
    src = edge_index[0].astype(jnp.int32)
    dst = edge_index[1].astype(jnp.int32)
    x = x.astype(f32)

    bins, cnts = _sc_binning(src, dst)

    agg1 = _sc_agg1(x.reshape(-1), bins, cnts).reshape(N, 2)
    xcat8 = jnp.concatenate(
        [x, agg1, jnp.zeros((N, 4), f32)], axis=1)
    w8 = jnp.concatenate(
        [W1_root.T.astype(f32), W1_rel.T.astype(f32),
         jnp.zeros((4, FH), f32)], axis=0)
    bias1 = jnp.broadcast_to(b1.astype(f32)[None, :], (8, FH))
    h1 = _tc_layer1(xcat8, w8, bias1)

    agg2 = _sc_agg128(h1, bins, cnts)
    h2 = _tc_layer(agg2, h1, W2_rel.T.astype(f32), W2_root.T.astype(f32),
                   jnp.broadcast_to(b2.astype(f32)[None, :], (8, FH)))
    agg3 = _sc_agg128(h2, bins, cnts)
    h3 = _tc_layer(agg3, h2, W3_rel.T.astype(f32), W3_root.T.astype(f32),
                   jnp.broadcast_to(b3.astype(f32)[None, :], (8, FH)))

    xr = h3.reshape(B, NPG, FH)
    out, out_adj, lmat, emat = _tc_pool(
        s.astype(f32), adj.astype(f32), xr)
    link_loss = lmat[0, 0]
    ent_loss = emat[0, 0]
    return (out, out_adj, link_loss, ent_loss, pos)


# speedup vs baseline: 3.2474x; 3.2474x over previous
import functools

import jax
import jax.numpy as jnp
from jax import lax
from jax.experimental import pallas as pl
from jax.experimental.pallas import tpu as pltpu
from jax.experimental.pallas import tpu_sc as plsc

B = 512
NPG = 68
N = B * NPG
E = N * 16
FH = 128
C = 16
EPS = 1e-15

NC, NS, LANES = 2, 16, 16
NW = NC * NS
NB = 64
BR = N // NB
CAP = 544
EPT = E // NW
ECH = 4352
GB = 64
MULT, MSH = 61681, 25
DUMMY = BR << 16


def _c(v):
    return jnp.int32(v)


def _fori(n, body, unroll=None):
    if isinstance(n, int):
        return lax.fori_loop(0, n, body, 0, unroll=unroll)
    return lax.fori_loop(_c(0), n.astype(jnp.int32), body, 0)


def _mo(v, m):
    return pl.multiple_of(v, m)


def _wid():
    return (lax.axis_index("s").astype(jnp.int32) * _c(NC)
            + lax.axis_index("c").astype(jnp.int32))


def _vperm(v, idx):
    dnums = lax.GatherDimensionNumbers(
        offset_dims=(), collapsed_slice_dims=(0,), start_index_map=(0,))
    return lax.gather(v, idx[:, None], dnums, slice_sizes=(1,),
                      mode=lax.GatherScatterMode.PROMISE_IN_BOUNDS)


def _i32(v):
    return jnp.full((LANES,), v, jnp.int32)


def _bin_body(src_hbm, dst_hbm, bins_hbm, cnts_hbm, sbuf, dbuf, binsv, cntv):
    w = _wid()
    iota = lax.iota(jnp.int32, LANES)
    ones = _i32(1)
    dummy16 = _i32(DUMMY)
    zero16 = _i32(0)

    def initb(i, _):
        binsv[pl.ds(_mo(i * _c(16), 16), 16)] = dummy16
        return 0

    _fori(NB * CAP // 16, initb, unroll=8)
    for j in range(NB // 16):
        cntv[pl.ds(j * 16, 16)] = zero16

    shifts = [(jnp.maximum(iota - k, 0), iota >= k) for k in range(1, 16)]

    def edge_vreg(i, _):
        s16 = sbuf[pl.ds(_mo(i * _c(16), 16), 16)]
        d16 = dbuf[pl.ds(_mo(i * _c(16), 16), 16)]
        bkt = (d16 * MULT) >> MSH
        ldst = d16 - bkt * BR
        packed = s16 | (ldst << 16)
        base = plsc.load_gather(cntv, [bkt])
        rank = zero16
        for idxk, mk in shifts:
            sh = _vperm(bkt, idxk)
            rank = rank + jnp.where(mk & (sh == bkt), 1, 0).astype(jnp.int32)
        pos = jnp.minimum(base + rank, CAP - 1)
        plsc.store_scatter(binsv, [bkt * CAP + pos], packed)
        plsc.addupdate_scatter(cntv, [bkt], ones)
        return 0

    for ch in range(EPT // ECH):
        base_off = _mo(w * _c(EPT) + _c(ch * ECH), 64)
        pltpu.sync_copy(src_hbm.at[pl.ds(base_off, ECH)], sbuf)
        pltpu.sync_copy(dst_hbm.at[pl.ds(base_off, ECH)], dbuf)
        _fori(ECH // 16, edge_vreg)

    pltpu.sync_copy(binsv, bins_hbm.at[pl.ds(_mo(w * _c(NB * CAP), 16), NB * CAP)])
    pltpu.sync_copy(cntv, cnts_hbm.at[pl.ds(_mo(w * _c(NB), 16), NB)])


def _count_at(cbuf, iota, s, b):
    j = s * _c(NB) + b
    vec = cbuf[pl.ds(_mo((j >> _c(4)) << _c(4), 16), 16)]
    return jnp.sum(jnp.where(iota == (j & _c(15)), vec, 0).astype(jnp.int32),
                   dtype=jnp.int32)


def _agg_body(htab, bins_hbm, cnts_hbm, agg_hbm,
              acc_sh, merged, idx0, idx1, ldstb0, ldstb1, rows0, rows1,
              zbuf, cbuf, sem0, sem1, sems):
    w = _wid()
    sid = lax.axis_index("s").astype(jnp.int32)
    slab = sid * _c(BR + 1)
    iota = lax.iota(jnp.int32, LANES)
    pltpu.sync_copy(cnts_hbm, cbuf)
    zf = jnp.zeros((LANES,), jnp.float32)
    dummy16 = _i32(DUMMY)

    def zb(i, _):
        zbuf[i, pl.ds(0, 16)] = zf
        for j in range(1, FH // 16):
            zbuf[i, pl.ds(j * 16, 16)] = zf
        return 0

    _fori(GB, zb, unroll=4)

    def prep(k, idxr, ldr):
        for g in range(GB // 16):
            pk = merged[pl.ds(_mo(k * _c(GB) + _c(g * 16), 16), 16)]
            idxr[pl.ds(g * 16, 16)] = pk & 0xFFFF
            ldr[pl.ds(g * 16, 16)] = (pk >> 16) + slab

    for r in range(NB // NW):
        b = _c(r * NW) + w

        zd = []
        for t in range(8):
            zd.append(pltpu.async_copy(
                zbuf, acc_sh.at[pl.ds(slab + _c(t * GB), GB)], sems))
        zd.append(pltpu.async_copy(
            zbuf.at[pl.ds(0, (BR + 1) - 8 * GB)],
            acc_sh.at[pl.ds(slab + _c(8 * GB), (BR + 1) - 8 * GB)], sems))
        for d in zd:
            d.wait()

        descs = []
        for s in range(NW):
            descs.append(pltpu.async_copy(
                bins_hbm.at[pl.ds(_mo(_c(s * NB * CAP) + b * _c(CAP), 16),
                                  CAP)],
                merged.at[pl.ds(s * CAP, CAP)], sems))
        for d in descs:
            d.wait()

        cursor = _c(0)
        for s in range(NW):
            cnt = _count_at(cbuf, iota, _c(s), b)
            nv = (cnt + _c(15)) >> _c(4)
            cur_s = cursor

            def cp(k2, _, cur_s=cur_s, s=s):
                v = merged[pl.ds(_mo(_c(s * CAP) + k2 * _c(16), 16), 16)]
                merged[pl.ds(_mo(cur_s + k2 * _c(16), 16), 16)] = v
                return 0

            if s > 0:
                _fori(nv, cp)
            cursor = cursor + ((cnt + _c(15)) & _c(-16))

        for t in range(GB // 16):
            merged[pl.ds(_mo(cursor + _c(t * 16), 16), 16)] = dummy16

        nbm = (cursor + _c(GB - 1)) >> _c(6)
        npair = (nbm + _c(1)) >> _c(1)

        @pl.when(nbm > 0)
        def _():
            prep(_c(0), idx0, ldstb0)
            pltpu.async_copy(htab.at[idx0], rows0, sem0)

        def pair(pp, _):
            k0 = pp * _c(2)
            k1 = k0 + _c(1)

            @pl.when(k1 < nbm)
            def _():
                prep(k1, idx1, ldstb1)
                pltpu.async_copy(htab.at[idx1], rows1, sem1)

            pltpu.make_async_copy(htab.at[idx0], rows0, sem0).wait()
            pltpu.sync_copy(rows0, acc_sh.at[ldstb0], add=True)

            @pl.when(k1 + _c(1) < nbm)
            def _():
                prep(k1 + _c(1), idx0, ldstb0)
                pltpu.async_copy(htab.at[idx0], rows0, sem0)

            @pl.when(k1 < nbm)
            def _():
                pltpu.make_async_copy(htab.at[idx1], rows1, sem1).wait()
                pltpu.sync_copy(rows1, acc_sh.at[ldstb1], add=True)

            return 0

        _fori(npair, pair)
        pltpu.sync_copy(acc_sh.at[pl.ds(slab, BR)],
                        agg_hbm.at[pl.ds(_mo(b * _c(BR), 16), BR)])


def _agg1_body(xflat_hbm, bins_hbm, cnts_hbm, agg_hbm,
               xbuf, acc1, pbuf, cbuf):
    w = _wid()
    iota = lax.iota(jnp.int32, LANES)
    pltpu.sync_copy(cnts_hbm, cbuf)
    pltpu.sync_copy(xflat_hbm, xbuf)
    zf = jnp.zeros((LANES,), jnp.float32)

    def zv(i, _):
        acc1[pl.ds(_mo(i * _c(16), 16), 16)] = zf
        return 0

    for r in range(NB // NW):
        b = _c(r * NW) + w
        _fori(1104 // 16, zv)

        def src_list(s, _):
            cnt = _count_at(cbuf, iota, s, b)
            nv = (cnt + _c(15)) >> _c(4)
            pltpu.sync_copy(bins_hbm.at[pl.ds(_mo(s * _c(NB * CAP) + b * _c(CAP), 16), CAP)], pbuf)

            def vreg(k, _):
                pk = pbuf[pl.ds(_mo(k * _c(16), 16), 16)]
                s2 = (pk & 0xFFFF) * 2
                d2 = (pk >> 16) * 2
                v0 = plsc.load_gather(xbuf, [s2])
                v1 = plsc.load_gather(xbuf, [s2 + 1])
                plsc.addupdate_scatter(acc1, [d2], v0)
                plsc.addupdate_scatter(acc1, [d2 + 1], v1)
                return 0

            _fori(nv, vreg)
            return 0

        _fori(NW, src_list)
        pltpu.sync_copy(acc1.at[pl.ds(0, BR * 2)],
                        agg_hbm.at[pl.ds(_mo(b * _c(BR * 2), 16), BR * 2)])


def _sc_binning(src, dst):
    mesh = plsc.VectorSubcoreMesh(core_axis_name="c", subcore_axis_name="s")
    f = pl.kernel(
        _bin_body,
        compiler_params=pltpu.CompilerParams(needs_layout_passes=False),
        out_type=(
            jax.ShapeDtypeStruct((NW * NB * CAP,), jnp.int32),
            jax.ShapeDtypeStruct((NW * NB,), jnp.int32),
        ),
        mesh=mesh,
        scratch_types=[
            pltpu.VMEM((ECH,), jnp.int32),
            pltpu.VMEM((ECH,), jnp.int32),
            pltpu.VMEM((NB * CAP,), jnp.int32),
            pltpu.VMEM((NB,), jnp.int32),
        ],
    )
    return f(src, dst)


def _sc_agg128(htab, bins, cnts):
    mesh = plsc.VectorSubcoreMesh(core_axis_name="c", subcore_axis_name="s")
    f = pl.kernel(
        _agg_body,
        compiler_params=pltpu.CompilerParams(needs_layout_passes=False),
        out_type=jax.ShapeDtypeStruct((N, FH), jnp.float32),
        mesh=mesh,
        scratch_types=[
            pltpu.VMEM_SHARED((NS * (BR + 1), FH), jnp.float32),
            pltpu.VMEM((NW * CAP + GB,), jnp.int32),
            pltpu.VMEM((GB,), jnp.int32),
            pltpu.VMEM((GB,), jnp.int32),
            pltpu.VMEM((GB,), jnp.int32),
            pltpu.VMEM((GB,), jnp.int32),
            pltpu.VMEM((GB, FH), jnp.float32),
            pltpu.VMEM((GB, FH), jnp.float32),
            pltpu.VMEM((GB, FH), jnp.float32),
            pltpu.VMEM((NW * NB,), jnp.int32),
            pltpu.SemaphoreType.DMA,
            pltpu.SemaphoreType.DMA,
            pltpu.SemaphoreType.DMA,
        ],
    )
    return f(htab, bins, cnts)


def _sc_agg1(xflat, bins, cnts):
    mesh = plsc.VectorSubcoreMesh(core_axis_name="c", subcore_axis_name="s")
    f = pl.kernel(
        _agg1_body,
        compiler_params=pltpu.CompilerParams(needs_layout_passes=False),
        out_type=jax.ShapeDtypeStruct((N * 2,), jnp.float32),
        mesh=mesh,
        scratch_types=[
            pltpu.VMEM((N * 2,), jnp.float32),
            pltpu.VMEM((1104,), jnp.float32),
            pltpu.VMEM((CAP,), jnp.int32),
            pltpu.VMEM((NW * NB,), jnp.int32),
        ],
    )
    return f(xflat, bins, cnts)


_RB = 512


def _l1_body(xc_ref, w_ref, b_ref, out_ref):
    z = jnp.dot(xc_ref[...], w_ref[...], preferred_element_type=jnp.float32)
    out_ref[...] = jnp.maximum(z + b_ref[0:1, :], 0.0)


def _mm_body(agg_ref, h_ref, wr_ref, wro_ref, b_ref, out_ref):
    z = (jnp.dot(agg_ref[...], wr_ref[...], preferred_element_type=jnp.float32)
         + jnp.dot(h_ref[...], wro_ref[...], preferred_element_type=jnp.float32))
    out_ref[...] = jnp.maximum(z + b_ref[0:1, :], 0.0)


def _tc_layer1(xcat8, w8, bias8):
    grid = (N // _RB,)
    return pl.pallas_call(
        _l1_body,
        grid=grid,
        in_specs=[
            pl.BlockSpec((_RB, 8), lambda i: (i, 0)),
            pl.BlockSpec((8, FH), lambda i: (0, 0)),
            pl.BlockSpec((8, FH), lambda i: (0, 0)),
        ],
        out_specs=pl.BlockSpec((_RB, FH), lambda i: (i, 0)),
        out_shape=jax.ShapeDtypeStruct((N, FH), jnp.float32),
    )(xcat8, w8, bias8)


def _tc_layer(agg, h, wrT, wroT, bias8):
    grid = (N // _RB,)
    return pl.pallas_call(
        _mm_body,
        grid=grid,
        in_specs=[
            pl.BlockSpec((_RB, FH), lambda i: (i, 0)),
            pl.BlockSpec((_RB, FH), lambda i: (i, 0)),
            pl.BlockSpec((FH, FH), lambda i: (0, 0)),
            pl.BlockSpec((FH, FH), lambda i: (0, 0)),
            pl.BlockSpec((8, FH), lambda i: (0, 0)),
        ],
        out_specs=pl.BlockSpec((_RB, FH), lambda i: (i, 0)),
        out_shape=jax.ShapeDtypeStruct((N, FH), jnp.float32),
    )(agg, h, wrT, wroT, bias8)


_BB = 64


def _pool_body(s_ref, adj_ref, xr_ref, out_ref, oadj_ref, link_ref, ent_ref):
    i = pl.program_id(0)
    sb = s_ref[...]
    m = jnp.max(sb, axis=-1, keepdims=True)
    e = jnp.exp(sb - m)
    ss = e / jnp.sum(e, axis=-1, keepdims=True)
    xr = xr_ref[...]
    out_ref[...] = lax.dot_general(
        ss, xr, (((1,), (1,)), ((0,), (0,))), preferred_element_type=jnp.float32)
    adjb = adj_ref[...]
    asx = lax.dot_general(
        adjb, ss, (((2,), (1,)), ((0,), (0,))), preferred_element_type=jnp.float32)
    oadj_ref[...] = lax.dot_general(
        ss, asx, (((1,), (1,)), ((0,), (0,))), preferred_element_type=jnp.float32)
    sst = lax.dot_general(
        ss, ss, (((2,), (2,)), ((0,), (0,))), preferred_element_type=jnp.float32)
    link = adjb - sst
    lpart = jnp.sum(link * link)
    epart = jnp.sum(-ss * jnp.log(ss + EPS))
    r0 = lax.broadcasted_iota(jnp.int32, (8, 128), 0)
    c0 = lax.broadcasted_iota(jnp.int32, (8, 128), 1)
    mask00 = (r0 == 0) & (c0 == 0)

    @pl.when(i == 0)
    def _():
        link_ref[...] = jnp.zeros((8, 128), jnp.float32)
        ent_ref[...] = jnp.zeros((8, 128), jnp.float32)

    link_ref[...] = link_ref[...] + jnp.where(mask00, lpart, 0.0)
    ent_ref[...] = ent_ref[...] + jnp.where(mask00, epart, 0.0)

    @pl.when(i == pl.num_programs(0) - 1)
    def _():
        lv = link_ref[...]
        link_ref[...] = jnp.where(
            mask00, jnp.sqrt(lv) / float(B * NPG * NPG), lv)
        ev = ent_ref[...]
        ent_ref[...] = jnp.where(mask00, ev / float(B * NPG), ev)


def _tc_pool(s, adj, xr):
    grid = (B // _BB,)
    return pl.pallas_call(
        _pool_body,
        grid=grid,
        in_specs=[
            pl.BlockSpec((_BB, NPG, C), lambda i: (i, 0, 0)),
            pl.BlockSpec((_BB, NPG, NPG), lambda i: (i, 0, 0)),
            pl.BlockSpec((_BB, NPG, FH), lambda i: (i, 0, 0)),
        ],
        out_specs=[
            pl.BlockSpec((_BB, C, FH), lambda i: (i, 0, 0)),
            pl.BlockSpec((_BB, C, C), lambda i: (i, 0, 0)),
            pl.BlockSpec((8, 128), lambda i: (0, 0)),
            pl.BlockSpec((8, 128), lambda i: (0, 0)),
        ],
        out_shape=[
            jax.ShapeDtypeStruct((B, C, FH), jnp.float32),
            jax.ShapeDtypeStruct((B, C, C), jnp.float32),
            jax.ShapeDtypeStruct((8, 128), jnp.float32),
            jax.ShapeDtypeStruct((8, 128), jnp.float32),
        ],
    )(s, adj, xr)


def kernel(x, edge_index, adj, s, pos,
           W1_rel, b1, W1_root, W2_rel, b2, W2_root,
           W3_rel, b3, W3_root):
    with jax.enable_x64(False):
        return _kernel32(x, edge_index, adj, s, pos,
                         W1_rel, b1, W1_root, W2_rel, b2, W2_root,
                         W3_rel, b3, W3_root)


def _kernel32(x, edge_index, adj, s, pos,
              W1_rel, b1, W1_root, W2_rel, b2, W2_root,
              W3_rel, b3, W3_root):
    f32 = jnp.float32
    src = edge_index[0].astype(jnp.int32)
    dst = edge_index[1].astype(jnp.int32)
    x = x.astype(f32)

    bins, cnts = _sc_binning(src, dst)

    agg1 = _sc_agg1(x.reshape(-1), bins, cnts).reshape(N, 2)
    xcat8 = jnp.concatenate(
        [x, agg1, jnp.zeros((N, 4), f32)], axis=1)
    w8 = jnp.concatenate(
        [W1_root.T.astype(f32), W1_rel.T.astype(f32),
         jnp.zeros((4, FH), f32)], axis=0)
    bias1 = jnp.broadcast_to(b1.astype(f32)[None, :], (8, FH))
    h1 = _tc_layer1(xcat8, w8, bias1)

    agg2 = _sc_agg128(h1, bins, cnts)
    h2 = _tc_layer(agg2, h1, W2_rel.T.astype(f32), W2_root.T.astype(f32),
                   jnp.broadcast_to(b2.astype(f32)[None, :], (8, FH)))
    agg3 = _sc_agg128(h2, bins, cnts)
    h3 = _tc_layer(agg3, h2, W3_rel.T.astype(f32), W3_root.T.astype(f32),
                   jnp.broadcast_to(b3.astype(f32)[None, :], (8, FH)))

    xr = h3.reshape(B, NPG, FH)
    out, out_adj, lmat, emat = _tc_pool(
        s.astype(f32), adj.astype(f32), xr)
    link_loss = lmat[0, 0]
    ent_loss = emat[0, 0]
    return (out, out_adj, link_loss, ent_loss, pos)


# speedup vs baseline: 3.3556x; 1.0333x over previous
import functools

import jax
import jax.numpy as jnp
from jax import lax
from jax.experimental import pallas as pl
from jax.experimental.pallas import tpu as pltpu
from jax.experimental.pallas import tpu_sc as plsc

B = 512
NPG = 68
N = B * NPG
E = N * 16
FH = 128
C = 16
EPS = 1e-15

ND, NDSH = 8, 3
NC, NS, LANES = 2, 16, 16
NW = NC * NS
NB = 64
BR = N // NB
CAP = 544
EPT = E // NW
ECH = 4352
GB = 32
MULT, MSH = 61681, 25
DUMMY = BR << 16


def _c(v):
    return jnp.int32(v)


def _fori(n, body, unroll=None):
    if isinstance(n, int):
        return lax.fori_loop(0, n, body, 0, unroll=unroll)
    return lax.fori_loop(_c(0), n.astype(jnp.int32), body, 0)


def _mo(v, m):
    return pl.multiple_of(v, m)


def _wid():
    return (lax.axis_index("s").astype(jnp.int32) * _c(NC)
            + lax.axis_index("c").astype(jnp.int32))


def _vperm(v, idx):
    dnums = lax.GatherDimensionNumbers(
        offset_dims=(), collapsed_slice_dims=(0,), start_index_map=(0,))
    return lax.gather(v, idx[:, None], dnums, slice_sizes=(1,),
                      mode=lax.GatherScatterMode.PROMISE_IN_BOUNDS)


def _i32(v):
    return jnp.full((LANES,), v, jnp.int32)


def _bin_body(src_hbm, dst_hbm, bins_hbm, cnts_hbm, sbuf, dbuf, binsv, cntv):
    w = _wid()
    iota = lax.iota(jnp.int32, LANES)
    ones = _i32(1)
    dummy16 = _i32(DUMMY)
    zero16 = _i32(0)

    def initb(i, _):
        binsv[pl.ds(_mo(i * _c(16), 16), 16)] = dummy16
        return 0

    _fori(NB * CAP // 16, initb, unroll=8)
    for j in range(NB // 16):
        cntv[pl.ds(j * 16, 16)] = zero16

    shifts = [(jnp.maximum(iota - k, 0), iota >= k) for k in range(1, 16)]

    def edge_vreg(i, _):
        s16 = sbuf[pl.ds(_mo(i * _c(16), 16), 16)]
        d16 = dbuf[pl.ds(_mo(i * _c(16), 16), 16)]
        bkt = (d16 * MULT) >> MSH
        ldst = d16 - bkt * BR
        packed = s16 | (ldst << 16)
        base = plsc.load_gather(cntv, [bkt])
        rank = zero16
        for idxk, mk in shifts:
            sh = _vperm(bkt, idxk)
            rank = rank + jnp.where(mk & (sh == bkt), 1, 0).astype(jnp.int32)
        pos = jnp.minimum(base + rank, CAP - 1)
        plsc.store_scatter(binsv, [bkt * CAP + pos], packed)
        plsc.addupdate_scatter(cntv, [bkt], ones)
        return 0

    for ch in range(EPT // ECH):
        base_off = _mo(w * _c(EPT) + _c(ch * ECH), 64)
        pltpu.sync_copy(src_hbm.at[pl.ds(base_off, ECH)], sbuf)
        pltpu.sync_copy(dst_hbm.at[pl.ds(base_off, ECH)], dbuf)
        _fori(ECH // 16, edge_vreg)

    pltpu.sync_copy(binsv, bins_hbm.at[pl.ds(_mo(w * _c(NB * CAP), 16), NB * CAP)])
    pltpu.sync_copy(cntv, cnts_hbm.at[pl.ds(_mo(w * _c(NB), 16), NB)])


def _count_at(cbuf, iota, s, b):
    j = s * _c(NB) + b
    vec = cbuf[pl.ds(_mo((j >> _c(4)) << _c(4), 16), 16)]
    return jnp.sum(jnp.where(iota == (j & _c(15)), vec, 0).astype(jnp.int32),
                   dtype=jnp.int32)


def _agg_body(htab, bins_hbm, cnts_hbm, agg_hbm,
              acc_sh, merged,
              idxa, idxb2, idxc, idxd, idxe, idxf, idxg, idxh,
              lda, ldb, ldc, ldd, lde, ldf, ldg, ldh,
              ra, rb2, rc, rd, re, rf, rg, rh,
              zbuf, cbuf,
              ga, gb2, gc, gd, ge, gf, gg, gh, sems):
    idxs = [idxa, idxb2, idxc, idxd, idxe, idxf, idxg, idxh]
    ldsts = [lda, ldb, ldc, ldd, lde, ldf, ldg, ldh]
    rowss = [ra, rb2, rc, rd, re, rf, rg, rh]
    gsems = [ga, gb2, gc, gd, ge, gf, gg, gh]
    w = _wid()
    sid = lax.axis_index("s").astype(jnp.int32)
    slab = sid * _c(BR + 1)
    iota = lax.iota(jnp.int32, LANES)
    pltpu.sync_copy(cnts_hbm, cbuf)
    zf = jnp.zeros((LANES,), jnp.float32)
    dummy16 = _i32(DUMMY)

    def zb(i, _):
        zbuf[i, pl.ds(0, 16)] = zf
        for j in range(1, FH // 16):
            zbuf[i, pl.ds(j * 16, 16)] = zf
        return 0

    _fori(GB, zb, unroll=4)

    def prep(k, idxr, ldr):
        for g in range(GB // 16):
            pk = merged[pl.ds(_mo(k * _c(GB) + _c(g * 16), 16), 16)]
            idxr[pl.ds(g * 16, 16)] = pk & 0xFFFF
            ldr[pl.ds(g * 16, 16)] = (pk >> 16) + slab

    for r in range(NB // NW):
        b = _c(r * NW) + w

        zd = []
        for t in range(17):
            zd.append(pltpu.async_copy(
                zbuf, acc_sh.at[pl.ds(slab + _c(t * GB), GB)], sems))
        zd.append(pltpu.async_copy(
            zbuf.at[pl.ds(0, 1)],
            acc_sh.at[pl.ds(slab + _c(17 * GB), 1)], sems))
        for d in zd:
            d.wait()

        descs = []
        for s in range(NW):
            descs.append(pltpu.async_copy(
                bins_hbm.at[pl.ds(_mo(_c(s * NB * CAP) + b * _c(CAP), 16),
                                  CAP)],
                merged.at[pl.ds(s * CAP, CAP)], sems))
        for d in descs:
            d.wait()

        cursor = _c(0)
        for s in range(NW):
            cnt = _count_at(cbuf, iota, _c(s), b)
            nv = (cnt + _c(15)) >> _c(4)
            cur_s = cursor

            def cp(k2, _, cur_s=cur_s, s=s):
                v = merged[pl.ds(_mo(_c(s * CAP) + k2 * _c(16), 16), 16)]
                merged[pl.ds(_mo(cur_s + k2 * _c(16), 16), 16)] = v
                return 0

            if s > 0:
                _fori(nv, cp)
            cursor = cursor + ((cnt + _c(15)) & _c(-16))

        for t in range(GB // 16):
            merged[pl.ds(_mo(cursor + _c(t * 16), 16), 16)] = dummy16

        nbm = (cursor + _c(GB - 1)) >> _c(5)
        nquad = (nbm + _c(ND - 1)) >> _c(NDSH)

        for j in range(ND - 1):
            @pl.when(_c(j) < nbm)
            def _(j=j):
                prep(_c(j), idxs[j], ldsts[j])
                pltpu.async_copy(htab.at[idxs[j]], rowss[j], gsems[j])

        def quad(pp, _):
            for j in range(ND):
                k = pp * _c(ND) + _c(j)

                @pl.when(k < nbm)
                def _(j=j, k=k):
                    pltpu.make_async_copy(
                        htab.at[idxs[j]], rowss[j], gsems[j]).wait()
                    pltpu.sync_copy(rowss[j], acc_sh.at[ldsts[j]], add=True)
                    kn = k + _c(ND - 1)
                    j3 = (j + ND - 1) % ND

                    @pl.when(kn < nbm)
                    def _():
                        prep(kn, idxs[j3], ldsts[j3])
                        pltpu.async_copy(htab.at[idxs[j3]], rowss[j3],
                                         gsems[j3])
            return 0

        _fori(nquad, quad)
        pltpu.sync_copy(acc_sh.at[pl.ds(slab, BR)],
                        agg_hbm.at[pl.ds(_mo(b * _c(BR), 16), BR)])


def _agg1_body(xflat_hbm, bins_hbm, cnts_hbm, agg_hbm,
               xbuf, acc1, pbuf, cbuf):
    w = _wid()
    iota = lax.iota(jnp.int32, LANES)
    pltpu.sync_copy(cnts_hbm, cbuf)
    pltpu.sync_copy(xflat_hbm, xbuf)
    zf = jnp.zeros((LANES,), jnp.float32)

    def zv(i, _):
        acc1[pl.ds(_mo(i * _c(16), 16), 16)] = zf
        return 0

    for r in range(NB // NW):
        b = _c(r * NW) + w
        _fori(1104 // 16, zv)

        def src_list(s, _):
            cnt = _count_at(cbuf, iota, s, b)
            nv = (cnt + _c(15)) >> _c(4)
            pltpu.sync_copy(bins_hbm.at[pl.ds(_mo(s * _c(NB * CAP) + b * _c(CAP), 16), CAP)], pbuf)

            def vreg(k, _):
                pk = pbuf[pl.ds(_mo(k * _c(16), 16), 16)]
                s2 = (pk & 0xFFFF) * 2
                d2 = (pk >> 16) * 2
                v0 = plsc.load_gather(xbuf, [s2])
                v1 = plsc.load_gather(xbuf, [s2 + 1])
                plsc.addupdate_scatter(acc1, [d2], v0)
                plsc.addupdate_scatter(acc1, [d2 + 1], v1)
                return 0

            _fori(nv, vreg)
            return 0

        _fori(NW, src_list)
        pltpu.sync_copy(acc1.at[pl.ds(0, BR * 2)],
                        agg_hbm.at[pl.ds(_mo(b * _c(BR * 2), 16), BR * 2)])


def _sc_binning(src, dst):
    mesh = plsc.VectorSubcoreMesh(core_axis_name="c", subcore_axis_name="s")
    f = pl.kernel(
        _bin_body,
        compiler_params=pltpu.CompilerParams(needs_layout_passes=False),
        out_type=(
            jax.ShapeDtypeStruct((NW * NB * CAP,), jnp.int32),
            jax.ShapeDtypeStruct((NW * NB,), jnp.int32),
        ),
        mesh=mesh,
        scratch_types=[
            pltpu.VMEM((ECH,), jnp.int32),
            pltpu.VMEM((ECH,), jnp.int32),
            pltpu.VMEM((NB * CAP,), jnp.int32),
            pltpu.VMEM((NB,), jnp.int32),
        ],
    )
    return f(src, dst)


def _sc_agg128(htab, bins, cnts):
    mesh = plsc.VectorSubcoreMesh(core_axis_name="c", subcore_axis_name="s")
    f = pl.kernel(
        _agg_body,
        compiler_params=pltpu.CompilerParams(needs_layout_passes=False),
        out_type=jax.ShapeDtypeStruct((N, FH), jnp.float32),
        mesh=mesh,
        scratch_types=(
            [pltpu.VMEM_SHARED((NS * (BR + 1), FH), jnp.float32),
             pltpu.VMEM((NW * CAP + GB,), jnp.int32)]
            + [pltpu.VMEM((GB,), jnp.int32) for _ in range(16)]
            + [pltpu.VMEM((GB, FH), jnp.float32) for _ in range(9)]
            + [pltpu.VMEM((NW * NB,), jnp.int32)]
            + [pltpu.SemaphoreType.DMA for _ in range(9)]
        ),
    )
    return f(htab, bins, cnts)


def _sc_agg1(xflat, bins, cnts):
    mesh = plsc.VectorSubcoreMesh(core_axis_name="c", subcore_axis_name="s")
    f = pl.kernel(
        _agg1_body,
        compiler_params=pltpu.CompilerParams(needs_layout_passes=False),
        out_type=jax.ShapeDtypeStruct((N * 2,), jnp.float32),
        mesh=mesh,
        scratch_types=[
            pltpu.VMEM((N * 2,), jnp.float32),
            pltpu.VMEM((1104,), jnp.float32),
            pltpu.VMEM((CAP,), jnp.int32),
            pltpu.VMEM((NW * NB,), jnp.int32),
        ],
    )
    return f(xflat, bins, cnts)


_RB = 512


def _l1_body(xc_ref, w_ref, b_ref, out_ref):
    z = jnp.dot(xc_ref[...], w_ref[...], preferred_element_type=jnp.float32)
    out_ref[...] = jnp.maximum(z + b_ref[0:1, :], 0.0)


def _mm_body(agg_ref, h_ref, wr_ref, wro_ref, b_ref, out_ref):
    z = (jnp.dot(agg_ref[...], wr_ref[...], preferred_element_type=jnp.float32)
         + jnp.dot(h_ref[...], wro_ref[...], preferred_element_type=jnp.float32))
    out_ref[...] = jnp.maximum(z + b_ref[0:1, :], 0.0)


def _tc_layer1(xcat8, w8, bias8):
    grid = (N // _RB,)
    return pl.pallas_call(
        _l1_body,
        grid=grid,
        in_specs=[
            pl.BlockSpec((_RB, 8), lambda i: (i, 0)),
            pl.BlockSpec((8, FH), lambda i: (0, 0)),
            pl.BlockSpec((8, FH), lambda i: (0, 0)),
        ],
        out_specs=pl.BlockSpec((_RB, FH), lambda i: (i, 0)),
        out_shape=jax.ShapeDtypeStruct((N, FH), jnp.float32),
    )(xcat8, w8, bias8)


def _tc_layer(agg, h, wrT, wroT, bias8):
    grid = (N // _RB,)
    return pl.pallas_call(
        _mm_body,
        grid=grid,
        in_specs=[
            pl.BlockSpec((_RB, FH), lambda i: (i, 0)),
            pl.BlockSpec((_RB, FH), lambda i: (i, 0)),
            pl.BlockSpec((FH, FH), lambda i: (0, 0)),
            pl.BlockSpec((FH, FH), lambda i: (0, 0)),
            pl.BlockSpec((8, FH), lambda i: (0, 0)),
        ],
        out_specs=pl.BlockSpec((_RB, FH), lambda i: (i, 0)),
        out_shape=jax.ShapeDtypeStruct((N, FH), jnp.float32),
    )(agg, h, wrT, wroT, bias8)


_BB = 64


def _pool_body(s_ref, adj_ref, xr_ref, out_ref, oadj_ref, link_ref, ent_ref):
    i = pl.program_id(0)
    sb = s_ref[...]
    m = jnp.max(sb, axis=-1, keepdims=True)
    e = jnp.exp(sb - m)
    ss = e / jnp.sum(e, axis=-1, keepdims=True)
    xr = xr_ref[...]
    out_ref[...] = lax.dot_general(
        ss, xr, (((1,), (1,)), ((0,), (0,))), preferred_element_type=jnp.float32)
    adjb = adj_ref[...]
    asx = lax.dot_general(
        adjb, ss, (((2,), (1,)), ((0,), (0,))), preferred_element_type=jnp.float32)
    oadj_ref[...] = lax.dot_general(
        ss, asx, (((1,), (1,)), ((0,), (0,))), preferred_element_type=jnp.float32)
    sst = lax.dot_general(
        ss, ss, (((2,), (2,)), ((0,), (0,))), preferred_element_type=jnp.float32)
    link = adjb - sst
    lpart = jnp.sum(link * link)
    epart = jnp.sum(-ss * jnp.log(ss + EPS))
    r0 = lax.broadcasted_iota(jnp.int32, (8, 128), 0)
    c0 = lax.broadcasted_iota(jnp.int32, (8, 128), 1)
    mask00 = (r0 == 0) & (c0 == 0)

    @pl.when(i == 0)
    def _():
        link_ref[...] = jnp.zeros((8, 128), jnp.float32)
        ent_ref[...] = jnp.zeros((8, 128), jnp.float32)

    link_ref[...] = link_ref[...] + jnp.where(mask00, lpart, 0.0)
    ent_ref[...] = ent_ref[...] + jnp.where(mask00, epart, 0.0)

    @pl.when(i == pl.num_programs(0) - 1)
    def _():
        lv = link_ref[...]
        link_ref[...] = jnp.where(
            mask00, jnp.sqrt(lv) / float(B * NPG * NPG), lv)
        ev = ent_ref[...]
        ent_ref[...] = jnp.where(mask00, ev / float(B * NPG), ev)


def _tc_pool(s, adj, xr):
    grid = (B // _BB,)
    return pl.pallas_call(
        _pool_body,
        grid=grid,
        in_specs=[
            pl.BlockSpec((_BB, NPG, C), lambda i: (i, 0, 0)),
            pl.BlockSpec((_BB, NPG, NPG), lambda i: (i, 0, 0)),
            pl.BlockSpec((_BB, NPG, FH), lambda i: (i, 0, 0)),
        ],
        out_specs=[
            pl.BlockSpec((_BB, C, FH), lambda i: (i, 0, 0)),
            pl.BlockSpec((_BB, C, C), lambda i: (i, 0, 0)),
            pl.BlockSpec((8, 128), lambda i: (0, 0)),
            pl.BlockSpec((8, 128), lambda i: (0, 0)),
        ],
        out_shape=[
            jax.ShapeDtypeStruct((B, C, FH), jnp.float32),
            jax.ShapeDtypeStruct((B, C, C), jnp.float32),
            jax.ShapeDtypeStruct((8, 128), jnp.float32),
            jax.ShapeDtypeStruct((8, 128), jnp.float32),
        ],
    )(s, adj, xr)


def kernel(x, edge_index, adj, s, pos,
           W1_rel, b1, W1_root, W2_rel, b2, W2_root,
           W3_rel, b3, W3_root):
    with jax.enable_x64(False):
        return _kernel32(x, edge_index, adj, s, pos,
                         W1_rel, b1, W1_root, W2_rel, b2, W2_root,
                         W3_rel, b3, W3_root)


def _kernel32(x, edge_index, adj, s, pos,
              W1_rel, b1, W1_root, W2_rel, b2, W2_root,
              W3_rel, b3, W3_root):
    f32 = jnp.float32
    src = edge_index[0].astype(jnp.int32)
    dst = edge_index[1].astype(jnp.int32)
    x = x.astype(f32)

    bins, cnts = _sc_binning(src, dst)

    agg1 = _sc_agg1(x.reshape(-1), bins, cnts).reshape(N, 2)
    xcat8 = jnp.concatenate(
        [x, agg1, jnp.zeros((N, 4), f32)], axis=1)
    w8 = jnp.concatenate(
        [W1_root.T.astype(f32), W1_rel.T.astype(f32),
         jnp.zeros((4, FH), f32)], axis=0)
    bias1 = jnp.broadcast_to(b1.astype(f32)[None, :], (8, FH))
    h1 = _tc_layer1(xcat8, w8, bias1)

    agg2 = _sc_agg128(h1, bins, cnts)
    h2 = _tc_layer(agg2, h1, W2_rel.T.astype(f32), W2_root.T.astype(f32),
                   jnp.broadcast_to(b2.astype(f32)[None, :], (8, FH)))
    agg3 = _sc_agg128(h2, bins, cnts)
    h3 = _tc_layer(agg3, h2, W3_rel.T.astype(f32), W3_root.T.astype(f32),
                   jnp.broadcast_to(b3.astype(f32)[None, :], (8, FH)))

    xr = h3.reshape(B, NPG, FH)
    out, out_adj, lmat, emat = _tc_pool(
        s.astype(f32), adj.astype(f32), xr)
    link_loss = lmat[0, 0]
    ent_loss = emat[0, 0]
    return (out, out_adj, link_loss, ent_loss, pos)
